# baseline (speedup 1.0000x reference)
import functools

import jax
import jax.numpy as jnp
from jax.experimental import pallas as pl
from jax.experimental.pallas import tpu as pltpu

_B, _S, _D, _LD = 4, 8192, 1024, 48
_CHUNKS = 4


def _gelu_exact(v):
    return 0.5 * v * (1.0 + jax.lax.erf(v * 0.7071067811865476))


def _dma_body(x_ref, lat_ref, wnt_ref, bnt_ref, lnw_ref, lnb_ref,
              we1_ref, be1_ref, we2_ref, be2_ref, wout_ref, bout_ref,
              wg_ref, bg_ref, out_ref,
              xh, lat_v, wnt_v, bnt_v, lnw_v, lnb_v, we1_v, be1_v, we2_v,
              be2_v, wout_v, bout_v, wg_v, bg_v, uph,
              sem_tail, sem_in, sem_out):
    tail = _S - _LD
    c0 = (tail // _CHUNKS) // 8 * 8

    tail_copies = []
    for b in range(_B):
        row0 = b * _S + _LD
        off = 0
        for c in range(_CHUNKS):
            sz = c0 if c < _CHUNKS - 1 else tail - c0 * (_CHUNKS - 1)
            cp = pltpu.make_async_copy(
                x_ref.at[pl.ds(row0 + off, sz), :],
                out_ref.at[pl.ds(row0 + off, sz), :],
                sem_tail)
            cp.start()
            tail_copies.append(cp)
            off += sz

    stage = [
        pltpu.make_async_copy(s, d, sem_in) for s, d in (
            (lat_ref, lat_v), (wnt_ref, wnt_v), (bnt_ref, bnt_v),
            (lnw_ref, lnw_v), (lnb_ref, lnb_v), (we1_ref, we1_v),
            (be1_ref, be1_v), (we2_ref, we2_v), (be2_ref, be2_v),
            (wout_ref, wout_v), (bout_ref, bout_v), (wg_ref, wg_v),
            (bg_ref, bg_v))
    ] + [
        pltpu.make_async_copy(x_ref.at[pl.ds(b * _S, _LD), :],
                              xh.at[pl.ds(b * _LD, _LD), :], sem_in)
        for b in range(_B)
    ]
    for cp in stage:
        cp.start()
    for cp in stage:
        cp.wait()

    R = _B * _LD
    xl = xh[...]
    h = jnp.dot(xl, wnt_v[...], preferred_element_type=jnp.float32) + bnt_v[...]
    mu = jnp.mean(h, axis=-1, keepdims=True)
    var = jnp.mean((h - mu) ** 2, axis=-1, keepdims=True)
    h = (h - mu) / jnp.sqrt(var + 1e-5) * lnw_v[...] + lnb_v[...]
    h = _gelu_exact(h)

    lat = lat_v[...]
    lat = lat - jnp.max(lat, axis=-1, keepdims=True)
    e = jnp.exp(lat)
    adj = e / jnp.sum(e, axis=-1, keepdims=True)
    w_masked = jnp.where(adj > 0.01, adj, 0.0)
    wtile = jnp.tile(w_masked, (_B, _B))
    rid = jax.lax.broadcasted_iota(jnp.int32, (R, R), 0) // _LD
    cid = jax.lax.broadcasted_iota(jnp.int32, (R, R), 1) // _LD
    wbig = jnp.where(rid == cid, wtile, 0.0)
    wn = jnp.dot(wbig, h, preferred_element_type=jnp.float32)

    msg = (jnp.dot(h, we1_v[:_D, :], preferred_element_type=jnp.float32)
           + jnp.dot(wn, we1_v[_D:, :], preferred_element_type=jnp.float32)
           + be1_v[...])
    msg = _gelu_exact(msg)
    msg = jnp.dot(msg, we2_v[...], preferred_element_type=jnp.float32) + be2_v[...]

    g = jax.nn.sigmoid(
        jnp.dot(xl, wg_v[:_D, :], preferred_element_type=jnp.float32)
        + jnp.dot(msg, wg_v[_D:, :], preferred_element_type=jnp.float32)
        + bg_v[...])
    uph[...] = g * (jnp.dot(msg, wout_v[...], preferred_element_type=jnp.float32)
                    + bout_v[...]) + (1.0 - g) * xl

    head_out = [
        pltpu.make_async_copy(uph.at[pl.ds(b * _LD, _LD), :],
                              out_ref.at[pl.ds(b * _S, _LD), :], sem_out)
        for b in range(_B)
    ]
    for cp in head_out:
        cp.start()
    for cp in head_out:
        cp.wait()
    for cp in tail_copies:
        cp.wait()


@functools.partial(jax.jit, static_argnames=("interpret",))
def _run(x, lattice_weights, W_nt, b_nt, ln_w, ln_b, W_e1, b_e1, W_e2, b_e2,
         W_out, b_out, W_g, b_g, interpret=False):
    B, S, D = x.shape
    xf = x.reshape(B * S, D)
    anyspec = pl.BlockSpec(memory_space=pl.ANY)
    vmem = lambda a: pltpu.VMEM(a.shape, a.dtype)
    out = pl.pallas_call(
        _dma_body,
        in_specs=[anyspec] * 14,
        out_specs=anyspec,
        out_shape=jax.ShapeDtypeStruct((B * S, D), x.dtype),
        scratch_shapes=[
            pltpu.VMEM((_B * _LD, D), jnp.float32),
            vmem(lattice_weights), vmem(W_nt), vmem(b_nt), vmem(ln_w),
            vmem(ln_b), vmem(W_e1), vmem(b_e1), vmem(W_e2), vmem(b_e2),
            vmem(W_out), vmem(b_out), vmem(W_g), vmem(b_g),
            pltpu.VMEM((_B * _LD, D), jnp.float32),
            pltpu.SemaphoreType.DMA,
            pltpu.SemaphoreType.DMA,
            pltpu.SemaphoreType.DMA,
        ],
        compiler_params=pltpu.CompilerParams(
            vmem_limit_bytes=100 * 1024 * 1024),
        interpret=interpret,
    )(xf, lattice_weights, W_nt, b_nt, ln_w, ln_b, W_e1, b_e1, W_e2, b_e2,
      W_out, b_out, W_g, b_g)
    return out.reshape(B, S, D)


def kernel(x, lattice_weights, W_nt, b_nt, ln_w, ln_b, W_e1, b_e1, W_e2,
           b_e2, W_out, b_out, W_g, b_g):
    return _run(x, lattice_weights, W_nt, b_nt, ln_w, ln_b, W_e1, b_e1,
                W_e2, b_e2, W_out, b_out, W_g, b_g)


# speedup vs baseline: 37.7877x; 37.7877x over previous
import functools

import jax
import jax.numpy as jnp
from jax.experimental import pallas as pl
from jax.experimental.pallas import tpu as pltpu

_B, _S, _D, _LD = 4, 8192, 1024, 48
_TOK_BLK = 2048


def _gelu_exact(v):
    return 0.5 * v * (1.0 + jax.lax.erf(v * 0.7071067811865476))


def _block_body(x_ref, lat_ref, wnt_ref, bnt_ref, lnw_ref, lnb_ref,
                we1_ref, be1_ref, we2_ref, be2_ref, wout_ref, bout_ref,
                wg_ref, bg_ref, out_ref):
    out_ref[...] = x_ref[...]

    @pl.when(pl.program_id(0) % (_S // _TOK_BLK) == 0)
    def _compute():
        L = _LD
        xl = x_ref[:L, :]
        xb = xl.astype(jnp.bfloat16)
        h = jnp.dot(xb, wnt_ref[...], preferred_element_type=jnp.float32)
        h = h + bnt_ref[...]
        mu = jnp.mean(h, axis=-1, keepdims=True)
        var = jnp.mean((h - mu) ** 2, axis=-1, keepdims=True)
        h = (h - mu) / jnp.sqrt(var + 1e-5) * lnw_ref[...] + lnb_ref[...]
        h = _gelu_exact(h)

        lat = lat_ref[...]
        lat = lat - jnp.max(lat, axis=-1, keepdims=True)
        e = jnp.exp(lat)
        adj = e / jnp.sum(e, axis=-1, keepdims=True)
        w_masked = jnp.where(adj > 0.01, adj, 0.0)
        wn = jnp.dot(w_masked, h, preferred_element_type=jnp.float32)

        hb = h.astype(jnp.bfloat16)
        wnb = wn.astype(jnp.bfloat16)
        msg = (jnp.dot(hb, we1_ref[:_D, :], preferred_element_type=jnp.float32)
               + jnp.dot(wnb, we1_ref[_D:, :], preferred_element_type=jnp.float32)
               + be1_ref[...])
        msg = _gelu_exact(msg)
        mb = msg.astype(jnp.bfloat16)
        msg = jnp.dot(mb, we2_ref[...], preferred_element_type=jnp.float32) + be2_ref[...]
        mb = msg.astype(jnp.bfloat16)

        g = jax.nn.sigmoid(
            jnp.dot(xb, wg_ref[:_D, :], preferred_element_type=jnp.float32)
            + jnp.dot(mb, wg_ref[_D:, :], preferred_element_type=jnp.float32)
            + bg_ref[...])
        upd = g * (jnp.dot(mb, wout_ref[...], preferred_element_type=jnp.float32)
                   + bout_ref[...]) + (1.0 - g) * xl
        out_ref[:L, :] = upd


@functools.partial(jax.jit, static_argnames=("interpret",))
def _run(x, lattice_weights, W_nt, b_nt, ln_w, ln_b, W_e1, b_e1, W_e2, b_e2,
         W_out, b_out, W_g, b_g, interpret=False):
    B, S, D = x.shape
    xf = x.reshape(B * S, D)
    bf = jnp.bfloat16
    W_nt, W_e1, W_e2, W_out, W_g = (w.astype(bf) for w in (W_nt, W_e1, W_e2, W_out, W_g))
    grid = (B * S // _TOK_BLK,)
    full = lambda a: pl.BlockSpec(a.shape, lambda t: (0,) * a.ndim)
    out = pl.pallas_call(
        _block_body,
        grid=grid,
        in_specs=[
            pl.BlockSpec((_TOK_BLK, D), lambda t: (t, 0)),
            full(lattice_weights), full(W_nt), full(b_nt), full(ln_w),
            full(ln_b), full(W_e1), full(b_e1), full(W_e2), full(b_e2),
            full(W_out), full(b_out), full(W_g), full(b_g),
        ],
        out_specs=pl.BlockSpec((_TOK_BLK, D), lambda t: (t, 0)),
        out_shape=jax.ShapeDtypeStruct((B * S, D), x.dtype),
        compiler_params=pltpu.CompilerParams(
            dimension_semantics=("arbitrary",),
            vmem_limit_bytes=116 * 1024 * 1024),
        interpret=interpret,
    )(xf, lattice_weights, W_nt, b_nt, ln_w, ln_b, W_e1, b_e1, W_e2, b_e2,
      W_out, b_out, W_g, b_g)
    return out.reshape(B, S, D)


def kernel(x, lattice_weights, W_nt, b_nt, ln_w, ln_b, W_e1, b_e1, W_e2,
           b_e2, W_out, b_out, W_g, b_g):
    return _run(x, lattice_weights, W_nt, b_nt, ln_w, ln_b, W_e1, b_e1,
                W_e2, b_e2, W_out, b_out, W_g, b_g)


# speedup vs baseline: 41.4348x; 1.0965x over previous
import functools

import jax
import jax.numpy as jnp
from jax.experimental import pallas as pl
from jax.experimental.pallas import tpu as pltpu

_B, _S, _D, _LD = 4, 8192, 1024, 48
_TOK_BLK = 2048


def _gelu_exact(v):
    return 0.5 * v * (1.0 + jax.lax.erf(v * 0.7071067811865476))


def _block_body(xh_ref, x_ref, lat_ref, wnt_ref, bnt_ref, lnw_ref, lnb_ref,
                we1_ref, be1_ref, we2_ref, be2_ref, wout_ref, bout_ref,
                wg_ref, bg_ref, out_ref, uph):
    out_ref[...] = x_ref[...]

    t = pl.program_id(0)
    steps_per_batch = _S // _TOK_BLK

    @pl.when(t == 0)
    def _compute():
        R = _B * _LD
        xl = xh_ref[...]
        h = jnp.dot(xl, wnt_ref[...], preferred_element_type=jnp.float32)
        h = h + bnt_ref[...]
        mu = jnp.mean(h, axis=-1, keepdims=True)
        var = jnp.mean((h - mu) ** 2, axis=-1, keepdims=True)
        h = (h - mu) / jnp.sqrt(var + 1e-5) * lnw_ref[...] + lnb_ref[...]
        h = _gelu_exact(h)

        lat = lat_ref[...]
        lat = lat - jnp.max(lat, axis=-1, keepdims=True)
        e = jnp.exp(lat)
        adj = e / jnp.sum(e, axis=-1, keepdims=True)
        w_masked = jnp.where(adj > 0.01, adj, 0.0)
        wtile = jnp.tile(w_masked, (_B, _B))
        rid = jax.lax.broadcasted_iota(jnp.int32, (R, R), 0) // _LD
        cid = jax.lax.broadcasted_iota(jnp.int32, (R, R), 1) // _LD
        wbig = jnp.where(rid == cid, wtile, 0.0)
        wn = jnp.dot(wbig, h, preferred_element_type=jnp.float32)

        msg = (jnp.dot(h, we1_ref[:_D, :], preferred_element_type=jnp.float32)
               + jnp.dot(wn, we1_ref[_D:, :], preferred_element_type=jnp.float32)
               + be1_ref[...])
        msg = _gelu_exact(msg)
        msg = jnp.dot(msg, we2_ref[...], preferred_element_type=jnp.float32) + be2_ref[...]

        g = jax.nn.sigmoid(
            jnp.dot(xl, wg_ref[:_D, :], preferred_element_type=jnp.float32)
            + jnp.dot(msg, wg_ref[_D:, :], preferred_element_type=jnp.float32)
            + bg_ref[...])
        uph[...] = g * (jnp.dot(msg, wout_ref[...], preferred_element_type=jnp.float32)
                        + bout_ref[...]) + (1.0 - g) * xl

    @pl.when(t % steps_per_batch == 0)
    def _splice():
        b = t // steps_per_batch
        out_ref[:_LD, :] = uph[pl.ds(b * _LD, _LD), :]


@functools.partial(jax.jit, static_argnames=("interpret",))
def _run(x, lattice_weights, W_nt, b_nt, ln_w, ln_b, W_e1, b_e1, W_e2, b_e2,
         W_out, b_out, W_g, b_g, interpret=False):
    B, S, D = x.shape
    xf = x.reshape(B * S, D)
    xh = x[:, :_LD, :].reshape(B * _LD, D)
    grid = (B * S // _TOK_BLK,)
    full = lambda a: pl.BlockSpec(a.shape, lambda t: (0,) * a.ndim)
    out = pl.pallas_call(
        _block_body,
        grid=grid,
        in_specs=[
            full(xh),
            pl.BlockSpec((_TOK_BLK, D), lambda t: (t, 0)),
            full(lattice_weights), full(W_nt), full(b_nt), full(ln_w),
            full(ln_b), full(W_e1), full(b_e1), full(W_e2), full(b_e2),
            full(W_out), full(b_out), full(W_g), full(b_g),
        ],
        out_specs=pl.BlockSpec((_TOK_BLK, D), lambda t: (t, 0)),
        out_shape=jax.ShapeDtypeStruct((B * S, D), x.dtype),
        scratch_shapes=[pltpu.VMEM((B * _LD, D), jnp.float32)],
        compiler_params=pltpu.CompilerParams(
            dimension_semantics=("arbitrary",),
            vmem_limit_bytes=100 * 1024 * 1024),
        interpret=interpret,
    )(xh, xf, lattice_weights, W_nt, b_nt, ln_w, ln_b, W_e1, b_e1, W_e2,
      b_e2, W_out, b_out, W_g, b_g)
    return out.reshape(B, S, D)


def kernel(x, lattice_weights, W_nt, b_nt, ln_w, ln_b, W_e1, b_e1, W_e2,
           b_e2, W_out, b_out, W_g, b_g):
    return _run(x, lattice_weights, W_nt, b_nt, ln_w, ln_b, W_e1, b_e1,
                W_e2, b_e2, W_out, b_out, W_g, b_g)
